# Initial kernel scaffold; baseline (speedup 1.0000x reference)
import jax
import jax.numpy as jnp
from jax.experimental import pallas as pl


def kernel(query, keys):
    raise NotImplementedError("write your pallas kernel here")


# speedup vs baseline: 2.9360x; 2.9360x over previous
import functools

import jax
import jax.numpy as jnp
from jax import lax
from jax.experimental import pallas as pl
from jax.experimental.pallas import tpu as pltpu
from jax.experimental.pallas import tpu_sc as plsc

_MEM = 512
_DIM = 512
_N = 8192
_T = 512
_INV_TEMP = 10.0
_L = 16


def _pass_a(q_ref, keys_ref,
            uqT_ref, smT_ref, eT_ref, gidx_ref, e1w_ref, denom_ref, s_ref,
            sep_ref, comp_ref,
            sume, acc_s, acc_m, acc_sep, acc_comp):
    b = pl.program_id(0)
    t = pl.program_id(1)
    first = (b == 0) & (t == 0)
    last = (b == pl.num_programs(0) - 1) & (t == pl.num_programs(1) - 1)

    @pl.when(first)
    def _init():
        sume[...] = jnp.zeros_like(sume)
        acc_s[...] = jnp.zeros_like(acc_s)
        acc_m[...] = jnp.full_like(acc_m, -jnp.inf)
        acc_sep[...] = jnp.zeros_like(acc_sep)
        acc_comp[...] = jnp.zeros_like(acc_comp)

    q = q_ref[0]
    keys = keys_ref[...]

    qss = jnp.sum(q * q, axis=0, keepdims=True)
    qnorm = jnp.maximum(jnp.sqrt(qss), 1e-12)
    qn = q / qnorm
    q2ss = jnp.sum(qn * qn, axis=0, keepdims=True)
    n1 = jnp.maximum(jnp.sqrt(q2ss), 1e-12)
    qn2 = qn / n1

    kss = jnp.sum(keys * keys, axis=1, keepdims=True)
    knorm = jnp.maximum(jnp.sqrt(kss), 1e-12)
    ksum = jnp.sum(keys, axis=1, keepdims=True)
    mn = keys / knorm

    sc = jax.lax.dot_general(mn, qn2, (((1,), (0,)), ((), ())),
                             preferred_element_type=jnp.float32)
    raw = sc * (knorm * n1)

    e = jnp.exp(sc * _INV_TEMP)
    eT_ref[0] = e
    colsum = jnp.sum(e, axis=0, keepdims=True)
    smT = e / colsum
    smT_ref[0] = smT

    acc_s[...] += jnp.sum(e, axis=1, keepdims=True)
    acc_m[...] = jnp.maximum(acc_m[...], jnp.max(sc, axis=1, keepdims=True))

    cmT = jax.lax.dot_general(keys, smT, (((0,), (0,)), ((), ())),
                              preferred_element_type=jnp.float32)
    uqT_ref[0, :_DIM, :] = qn
    uqT_ref[0, _DIM:, :] = cmT

    iota = jax.lax.broadcasted_iota(jnp.int32, sc.shape, 0)
    m1 = jnp.max(sc, axis=0, keepdims=True)
    idx1 = jnp.min(jnp.where(sc == m1, iota, _MEM), axis=0, keepdims=True)
    oh1 = iota == idx1
    sc2 = jnp.where(oh1, -jnp.inf, sc)
    m2 = jnp.max(sc2, axis=0, keepdims=True)
    idx2 = jnp.min(jnp.where(sc2 == m2, iota, _MEM), axis=0, keepdims=True)
    oh2 = iota == idx2

    zero = jnp.zeros_like(sc)
    raw1 = jnp.sum(jnp.where(oh1, raw, zero), axis=0, keepdims=True)
    raw2 = jnp.sum(jnp.where(oh2, raw, zero), axis=0, keepdims=True)
    kss1 = jnp.sum(jnp.where(oh1, kss, zero), axis=0, keepdims=True)
    kss2 = jnp.sum(jnp.where(oh2, kss, zero), axis=0, keepdims=True)
    ksum1 = jnp.sum(jnp.where(oh1, ksum, zero), axis=0, keepdims=True)
    ksum2 = jnp.sum(jnp.where(oh2, ksum, zero), axis=0, keepdims=True)
    qrss = q2ss
    qrsum = jnp.sum(qn, axis=0, keepdims=True)
    d1sq = qrss - 2.0 * raw1 + kss1
    d2sq = qrss - 2.0 * raw2 + kss2
    epsd = _DIM * 1e-12
    dp = jnp.sqrt(jnp.maximum(d1sq + 2e-6 * (qrsum - ksum1) + epsd, 0.0))
    dn = jnp.sqrt(jnp.maximum(d2sq + 2e-6 * (qrsum - ksum2) + epsd, 0.0))
    acc_sep[...] += jnp.sum(jnp.maximum(dp - dn + 1.0, 0.0)).reshape(1, 1)
    acc_comp[...] += jnp.sum(d1sq).reshape(1, 1)

    e1 = jnp.sum(jnp.where(oh1, e, zero), axis=0, keepdims=True)
    gidx_ref[0] = idx1
    e1w_ref[0] = e1
    ohw = jnp.where(oh1, e1, zero)
    sume[...] += jnp.sum(ohw, axis=1, keepdims=True)

    @pl.when(last)
    def _finalize():
        s = acc_s[...]
        e_max = jnp.exp(acc_m[...] * _INV_TEMP)
        denom_ref[...] = sume[...] + 1e-8 * (e_max + 1e-8 * s)
        s_ref[...] = s
        sep_ref[...] = acc_sep[...] / float(_N)
        comp_ref[...] = acc_comp[...] / float(_N * _DIM)


def _pass_b(e_ref, s_ref, sq_ref):
    sq_ref[0] = e_ref[0] * (1.0 / s_ref[...])


def _sc_segsum_body(uqT_hbm, g_hbm, e_hbm, out_hbm, qbuf, gbuf, ebuf, acc):
    c = lax.axis_index("c")
    s = lax.axis_index("s")
    wid = s * 2 + c
    dbase = wid * _L

    def zero(i, _):
        acc[pl.ds(i * _L, _L)] = jnp.zeros((_L,), jnp.float32)
        return 0
    lax.fori_loop(0, _MEM, zero, 0)

    iot = lax.iota(jnp.int32, _L)
    for b in range(8):
        pltpu.sync_copy(uqT_hbm.at[b, pl.ds(dbase, _L), :], qbuf)
        pltpu.sync_copy(g_hbm.at[b, 0, :], gbuf)
        pltpu.sync_copy(e_hbm.at[b, 0, :], ebuf)

        def body(j, _):
            jf = jnp.full((_L,), j, jnp.int32)
            gs = plsc.load_gather(gbuf, [jf])
            es = plsc.load_gather(ebuf, [jf])
            qv = plsc.load_gather(qbuf, [iot, jf])
            plsc.addupdate_scatter(acc, [gs * _L + iot], es * qv)
            return 0
        lax.fori_loop(0, qbuf.shape[1], body, 0)

    pltpu.sync_copy(acc, out_hbm.at[wid])


def _pass_c(sumeq_ref, denom_ref, keys_ref, um_ref):
    qu = sumeq_ref[...] / denom_ref[...]
    um = qu + keys_ref[...]
    umn = jnp.maximum(jnp.sqrt(jnp.sum(um * um, axis=1, keepdims=True)), 1e-12)
    um_ref[...] = um / umn


@jax.jit
def kernel(query, keys):
    bs = query.shape[0]
    hw = query.shape[2] * query.shape[3]
    q3 = query.reshape(bs, _DIM, hw)
    nt = hw // _T
    f32 = jnp.float32

    grid = (bs, nt)
    out_shapes = [
        jax.ShapeDtypeStruct((bs, 2 * _DIM, hw), f32),
        jax.ShapeDtypeStruct((bs, _MEM, hw), f32),
        jax.ShapeDtypeStruct((bs, _MEM, hw), f32),
        jax.ShapeDtypeStruct((bs, 1, hw), jnp.int32),
        jax.ShapeDtypeStruct((bs, 1, hw), f32),
        jax.ShapeDtypeStruct((_MEM, 1), f32),
        jax.ShapeDtypeStruct((_MEM, 1), f32),
        jax.ShapeDtypeStruct((1, 1), f32),
        jax.ShapeDtypeStruct((1, 1), f32),
    ]
    out_specs = [
        pl.BlockSpec((1, 2 * _DIM, _T), lambda b, t: (b, 0, t)),
        pl.BlockSpec((1, _MEM, _T), lambda b, t: (b, 0, t)),
        pl.BlockSpec((1, _MEM, _T), lambda b, t: (b, 0, t)),
        pl.BlockSpec((1, 1, _T), lambda b, t: (b, 0, t)),
        pl.BlockSpec((1, 1, _T), lambda b, t: (b, 0, t)),
        pl.BlockSpec((_MEM, 1), lambda b, t: (0, 0)),
        pl.BlockSpec((_MEM, 1), lambda b, t: (0, 0)),
        pl.BlockSpec((1, 1), lambda b, t: (0, 0)),
        pl.BlockSpec((1, 1), lambda b, t: (0, 0)),
    ]
    (uqT, smT, eT, gidx, e1w, denom, s, sep, comp) = pl.pallas_call(
        _pass_a,
        grid=grid,
        in_specs=[
            pl.BlockSpec((1, _DIM, _T), lambda b, t: (b, 0, t)),
            pl.BlockSpec((_MEM, _DIM), lambda b, t: (0, 0)),
        ],
        out_specs=out_specs,
        out_shape=out_shapes,
        scratch_shapes=[
            pltpu.VMEM((_MEM, 1), f32),
            pltpu.VMEM((_MEM, 1), f32),
            pltpu.VMEM((_MEM, 1), f32),
            pltpu.VMEM((1, 1), f32),
            pltpu.VMEM((1, 1), f32),
        ],
        compiler_params=pltpu.CompilerParams(
            dimension_semantics=("arbitrary", "arbitrary")),
    )(q3, keys)

    sqT = pl.pallas_call(
        _pass_b,
        grid=(bs, nt),
        in_specs=[
            pl.BlockSpec((1, _MEM, _T), lambda b, t: (b, 0, t)),
            pl.BlockSpec((_MEM, 1), lambda b, t: (0, 0)),
        ],
        out_specs=pl.BlockSpec((1, _MEM, _T), lambda b, t: (b, 0, t)),
        out_shape=jax.ShapeDtypeStruct((bs, _MEM, hw), f32),
        compiler_params=pltpu.CompilerParams(
            dimension_semantics=("arbitrary", "arbitrary")),
    )(eT, s)

    mesh = plsc.VectorSubcoreMesh(core_axis_name="c", subcore_axis_name="s")
    sumeq_rows = pl.kernel(
        _sc_segsum_body,
        mesh=mesh,
        out_type=jax.ShapeDtypeStruct((32, _MEM * _L), f32),
        scratch_types=[
            pltpu.VMEM((_L, hw), f32),
            pltpu.VMEM((hw,), jnp.int32),
            pltpu.VMEM((hw,), f32),
            pltpu.VMEM((_MEM * _L,), f32),
        ],
        compiler_params=pltpu.CompilerParams(needs_layout_passes=False),
    )(uqT, gidx, e1w)
    sumeq = sumeq_rows.reshape(32, _MEM, _L).transpose(1, 0, 2).reshape(
        _MEM, _DIM)

    um = pl.pallas_call(
        _pass_c,
        in_specs=[
            pl.BlockSpec((_MEM, _DIM), lambda: (0, 0)),
            pl.BlockSpec((_MEM, 1), lambda: (0, 0)),
            pl.BlockSpec((_MEM, _DIM), lambda: (0, 0)),
        ],
        out_specs=pl.BlockSpec((_MEM, _DIM), lambda: (0, 0)),
        out_shape=jax.ShapeDtypeStruct((_MEM, _DIM), f32),
    )(sumeq, denom, keys)

    n = bs * hw
    uq = uqT.reshape(bs, 2 * _DIM, query.shape[2], query.shape[3])
    sm = smT.transpose(0, 2, 1).reshape(n, _MEM)
    sq = sqT.transpose(0, 2, 1).reshape(n, _MEM)
    return (uq, um, sq, sm, sep.reshape(()), comp.reshape(()))
